# Initial kernel scaffold; baseline (speedup 1.0000x reference)
import jax
import jax.numpy as jnp
from jax.experimental import pallas as pl


def kernel(x, edge_index, W1, b1, W2, b2):
    raise NotImplementedError("write your pallas kernel here")


# speedup vs baseline: 17.6965x; 17.6965x over previous
import functools

import jax
import jax.numpy as jnp
from jax import lax
from jax.experimental import pallas as pl
from jax.experimental.pallas import tpu as pltpu
from jax.experimental.pallas import tpu_sc as plsc

N_NODES = 10000
N_EDGES = 320000
IN_CH = 128
HID_CH = 128
OUT_CH = 64

NC = 2
NS = 16
NW = NC * NS
EPT = N_EDGES // NW
CB = 80
NCH = EPT // CB
N_PAD = 10240
RPT = N_PAD // NS
DEGW = 128

_MESH = plsc.VectorSubcoreMesh(core_axis_name="c", subcore_axis_name="s")


@functools.partial(
    pl.kernel,
    out_type=jax.ShapeDtypeStruct((NC, N_PAD, DEGW), jnp.float32),
    mesh=_MESH,
    scratch_types=[
        pltpu.VMEM((NCH, CB), jnp.int32),
        pltpu.VMEM((CB, DEGW), jnp.float32),
        pltpu.VMEM_SHARED((N_PAD, DEGW), jnp.float32),
    ],
)
def _deg_kernel(dst_hbm, ones_hbm, zeros_hbm, out_hbm, dstv, onesv, acc):
    c = lax.axis_index("c")
    s = lax.axis_index("s")
    wid = c * NS + s
    pltpu.sync_copy(dst_hbm.at[wid], dstv)
    pltpu.sync_copy(ones_hbm, onesv)
    pltpu.sync_copy(zeros_hbm.at[pl.ds(s * RPT, RPT)],
                    acc.at[pl.ds(s * RPT, RPT)])
    plsc.subcore_barrier()

    def body(j, carry):
        pltpu.sync_copy(onesv, acc.at[dstv.at[j]], add=True)
        return carry

    lax.fori_loop(0, NCH, body, 0)
    plsc.subcore_barrier()
    pltpu.sync_copy(acc.at[pl.ds(s * RPT, RPT)],
                    out_hbm.at[c, pl.ds(s * RPT, RPT)])


def _make_agg(D):
    @functools.partial(
        pl.kernel,
        out_type=jax.ShapeDtypeStruct((NC, N_PAD, D), jnp.float32),
        mesh=_MESH,
        scratch_types=[
            pltpu.VMEM((NCH, CB), jnp.int32),
            pltpu.VMEM((NCH, CB), jnp.int32),
            pltpu.VMEM((CB, D), jnp.float32),
            pltpu.VMEM_SHARED((N_PAD, D), jnp.float32),
            pltpu.SemaphoreType.DMA,
        ],
    )
    def agg(src_hbm, dst_hbm, table_hbm, zeros_hbm, out_hbm,
            srcv, dstv, rowv, acc, sem):
        c = lax.axis_index("c")
        s = lax.axis_index("s")
        wid = c * NS + s
        pltpu.sync_copy(src_hbm.at[wid], srcv)
        pltpu.sync_copy(dst_hbm.at[wid], dstv)
        pltpu.sync_copy(zeros_hbm.at[pl.ds(s * RPT, RPT)],
                        acc.at[pl.ds(s * RPT, RPT)])
        plsc.subcore_barrier()

        def body(j, carry):
            pltpu.async_copy(table_hbm.at[srcv.at[j]], rowv, sem).wait()
            pltpu.sync_copy(rowv, acc.at[dstv.at[j]], add=True)
            return carry

        lax.fori_loop(0, NCH, body, 0)
        plsc.subcore_barrier()
        pltpu.sync_copy(acc.at[pl.ds(s * RPT, RPT)],
                        out_hbm.at[c, pl.ds(s * RPT, RPT)])

    return agg


_agg_hid = _make_agg(HID_CH)


RB = 2000
TCG = N_NODES // RB


def _norm(deg_blk):
    deg = deg_blk[0] + deg_blk[1] + 1.0
    return lax.rsqrt(deg)[:, 0:1]


def _tca_body(x_ref, w_ref, deg_ref, y_ref, ys_ref):
    y = jnp.dot(x_ref[...], w_ref[...], preferred_element_type=jnp.float32)
    n = _norm(deg_ref[...])
    y_ref[...] = y
    ys_ref[...] = y * n


def _tcb_body(p1_ref, y1_ref, b1_ref, deg_ref, h_ref, hs_ref):
    n = _norm(deg_ref[...])
    agg = p1_ref[0] + p1_ref[1]
    h = jnp.maximum(n * agg + (n * n) * y1_ref[...] + b1_ref[...], 0.0)
    h_ref[...] = h
    hs_ref[...] = h * n


def _tcc_body(p2_ref, h_ref, b2_ref, deg_ref, w2_ref, o_ref):
    n = _norm(deg_ref[...])
    z = n * (p2_ref[0] + p2_ref[1]) + (n * n) * h_ref[...]
    o_ref[...] = (jnp.dot(z, w2_ref[...], preferred_element_type=jnp.float32)
                  + b2_ref[...])


def _row_spec(d):
    return pl.BlockSpec((RB, d), lambda i: (i, 0))


def _part_spec(d):
    return pl.BlockSpec((2, RB, d), lambda i: (0, i, 0))


def _full_spec(r, d):
    return pl.BlockSpec((r, d), lambda i: (0, 0))


_tca = pl.pallas_call(
    _tca_body,
    grid=(TCG,),
    in_specs=[_row_spec(IN_CH), _full_spec(IN_CH, HID_CH), _part_spec(DEGW)],
    out_specs=[_row_spec(HID_CH), _row_spec(HID_CH)],
    out_shape=[jax.ShapeDtypeStruct((N_NODES, HID_CH), jnp.float32)] * 2,
)

_tcb = pl.pallas_call(
    _tcb_body,
    grid=(TCG,),
    in_specs=[_part_spec(HID_CH), _row_spec(HID_CH), _full_spec(1, HID_CH),
              _part_spec(DEGW)],
    out_specs=[_row_spec(HID_CH), _row_spec(HID_CH)],
    out_shape=[jax.ShapeDtypeStruct((N_NODES, HID_CH), jnp.float32)] * 2,
)

_tcc = pl.pallas_call(
    _tcc_body,
    grid=(TCG,),
    in_specs=[_part_spec(HID_CH), _row_spec(HID_CH), _full_spec(1, OUT_CH),
              _part_spec(DEGW), _full_spec(HID_CH, OUT_CH)],
    out_specs=_row_spec(OUT_CH),
    out_shape=jax.ShapeDtypeStruct((N_NODES, OUT_CH), jnp.float32),
)


def kernel(x, edge_index, W1, b1, W2, b2):
    ei = edge_index.astype(jnp.int32)
    src3 = ei[0].reshape(NW, NCH, CB)
    dst3 = ei[1].reshape(NW, NCH, CB)
    ones_h = jnp.ones((CB, DEGW), jnp.float32)
    zeros_deg = jnp.zeros((N_PAD, DEGW), jnp.float32)
    zeros_hid = jnp.zeros((N_PAD, HID_CH), jnp.float32)

    deg2 = _deg_kernel(dst3, ones_h, zeros_deg)
    y1, y1s = _tca(x, W1, deg2)
    part1 = _agg_hid(src3, dst3, y1s, zeros_hid)
    h, hs = _tcb(part1, y1, b1.reshape(1, HID_CH), deg2)
    part2 = _agg_hid(src3, dst3, hs, zeros_hid)
    out = _tcc(part2, h, b2.reshape(1, OUT_CH), deg2, W2)
    return out
